# baseline (speedup 1.0000x reference)
import functools

import jax
import jax.numpy as jnp
from jax import lax
from jax.experimental import pallas as pl
from jax.experimental.pallas import tpu as pltpu
from jax.experimental.pallas import tpu_sc as plsc

N = 10000
E = 320000
G = 64
D = 128
NC = 2
NS = 16
CH = 128
NP = 10240
RPS = NP // NS
NCH = 160
IB = 32
ESUB = NCH * CH
EPAD = ESUB * NS
RB = 512
NPB = NP // RB


@functools.cache
def _sc_kernels():
    mesh = plsc.VectorSubcoreMesh(core_axis_name="c", subcore_axis_name="s",
                                  num_cores=NC, num_subcores=NS)

    @functools.partial(
        pl.kernel,
        out_type=jax.ShapeDtypeStruct((NC * NS * NP,), jnp.float32),
        mesh=mesh,
        scratch_types=[
            pltpu.VMEM((NCH, CH), jnp.int32),
            pltpu.VMEM((NP,), jnp.float32),
        ],
        compiler_params=pltpu.CompilerParams(needs_layout_passes=False),
    )
    def _sc_degree(dst_hbm, zeros1d_hbm, cnt_hbm, dst_v, cnt_v):
        cid = lax.axis_index("c")
        sid = lax.axis_index("s")
        wid = cid * NS + sid
        pltpu.sync_copy(zeros1d_hbm, cnt_v)
        pltpu.sync_copy(dst_hbm.at[pl.ds(wid * NCH, NCH)], dst_v)
        ones16 = jnp.ones((16,), jnp.float32)

        def step(e, carry):
            idx = dst_v[e >> 3, pl.ds((e & 7) * 16, 16)]
            plsc.addupdate_scatter(cnt_v, [idx], ones16)
            return carry

        lax.fori_loop(0, NCH * (CH // 16), step, 0)
        pltpu.sync_copy(cnt_v, cnt_hbm.at[pl.ds(wid * NP, NP)])

    def _make_conv(g):

        @functools.partial(
            pl.kernel,
            out_type=jax.ShapeDtypeStruct((NP, D), jnp.float32),
            mesh=mesh,
            scratch_types=[
                pltpu.VMEM((IB, CH), jnp.int32),
                pltpu.VMEM((IB, CH), jnp.int32),
                pltpu.VMEM((CH, D), jnp.float32),
                pltpu.VMEM((CH, D), jnp.float32),
                pltpu.SemaphoreType.DMA,
                pltpu.SemaphoreType.DMA,
                pltpu.SemaphoreType.DMA,
                pltpu.SemaphoreType.DMA,
                pltpu.SemaphoreType.DMA,
                pltpu.SemaphoreType.DMA,
                pltpu.VMEM_SHARED((NP, D), jnp.float32),
            ],
        )
        def _sc_conv(hn_hbm, src_hbm, dst_hbm, zeros_hbm, out_hbm,
                     src_v, dst_v, rows0, rows1, gs0a, gs0b, gs1a, gs1b,
                     ss0, ss1, acc):
            cid = lax.axis_index("c")
            sid = lax.axis_index("s")

            @pl.when(cid == g)
            def _():
                pltpu.sync_copy(zeros_hbm, acc.at[pl.ds(sid * RPS, RPS)])
                plsc.subcore_barrier()

                H = CH // 2

                def gather(j, buf, sa, sb):
                    pltpu.async_copy(hn_hbm.at[src_v.at[j, pl.ds(0, H)]],
                                     buf.at[pl.ds(0, H)], sa)
                    pltpu.async_copy(hn_hbm.at[src_v.at[j, pl.ds(H, H)]],
                                     buf.at[pl.ds(H, H)], sb)

                def gwait(buf, sa, sb):
                    pltpu.make_async_copy(
                        hn_hbm.at[src_v.at[0, pl.ds(0, H)]],
                        buf.at[pl.ds(0, H)], sa).wait()
                    pltpu.make_async_copy(
                        hn_hbm.at[src_v.at[0, pl.ds(H, H)]],
                        buf.at[pl.ds(H, H)], sb).wait()

                def block(b, carry):
                    base = sid * NCH + b * IB
                    pltpu.sync_copy(src_hbm.at[pl.ds(base, IB)], src_v)
                    pltpu.sync_copy(dst_hbm.at[pl.ds(base, IB)], dst_v)
                    gather(0, rows0, gs0a, gs0b)
                    gather(1, rows1, gs1a, gs1b)

                    def pair(i, c2):
                        j0 = 2 * i
                        j1 = j0 + 1
                        gwait(rows0, gs0a, gs0b)
                        s0 = pltpu.async_copy(rows0, acc.at[dst_v.at[j0]],
                                              ss0, add=True)
                        gwait(rows1, gs1a, gs1b)
                        s1 = pltpu.async_copy(rows1, acc.at[dst_v.at[j1]],
                                              ss1, add=True)

                        @pl.when(i + 1 < IB // 2)
                        def _():
                            s0.wait()
                            gather(j0 + 2, rows0, gs0a, gs0b)
                            s1.wait()
                            gather(j1 + 2, rows1, gs1a, gs1b)

                        @pl.when(i + 1 >= IB // 2)
                        def _():
                            s0.wait()
                            s1.wait()

                        return c2

                    lax.fori_loop(0, IB // 2, pair, 0)
                    return carry

                lax.fori_loop(0, NCH // IB, block, 0)
                plsc.subcore_barrier()
                pltpu.sync_copy(acc.at[pl.ds(sid * RPS, RPS)],
                                out_hbm.at[pl.ds(sid * RPS, RPS)])

        return _sc_conv

    return _sc_degree, _make_conv(0), _make_conv(1)


def _tc_hn1_body(cnt_ref, xi_ref, emb_ref, w1_ref, out_ref):
    deg = jnp.sum(cnt_ref[...], axis=1, keepdims=True)
    dinv = lax.rsqrt(deg + 1.0)
    oh = (xi_ref[...] == lax.broadcasted_iota(jnp.int32, (RB, 16), 1)
          ).astype(jnp.float32)
    ew = jnp.dot(emb_ref[...], w1_ref[...], preferred_element_type=jnp.float32)
    h = jnp.dot(oh, ew, preferred_element_type=jnp.float32)
    out_ref[...] = h * dinv


_tc_hn1 = pl.pallas_call(
    _tc_hn1_body,
    grid=(NPB,),
    in_specs=[
        pl.BlockSpec((RB, NS), lambda i: (i, 0)),
        pl.BlockSpec((RB, 1), lambda i: (i, 0)),
        pl.BlockSpec((16, D), lambda i: (0, 0)),
        pl.BlockSpec((D, D), lambda i: (0, 0)),
    ],
    out_specs=pl.BlockSpec((RB, D), lambda i: (i, 0)),
    out_shape=jax.ShapeDtypeStruct((NP, D), jnp.float32),
)


def _tc_hn2_body(s_ref, hn_ref, cnt_ref, b_ref, w_ref, out_ref):
    deg = jnp.sum(cnt_ref[...], axis=1, keepdims=True)
    dinv = lax.rsqrt(deg + 1.0)
    o = jnp.maximum((s_ref[...] + hn_ref[...]) * dinv + b_ref[...], 0.0)
    out_ref[...] = jnp.dot(o, w_ref[...],
                           preferred_element_type=jnp.float32) * dinv


_tc_hn2 = pl.pallas_call(
    _tc_hn2_body,
    grid=(NPB,),
    in_specs=[
        pl.BlockSpec((RB, D), lambda i: (i, 0)),
        pl.BlockSpec((RB, D), lambda i: (i, 0)),
        pl.BlockSpec((RB, NS), lambda i: (i, 0)),
        pl.BlockSpec((1, D), lambda i: (0, 0)),
        pl.BlockSpec((D, D), lambda i: (0, 0)),
    ],
    out_specs=pl.BlockSpec((RB, D), lambda i: (i, 0)),
    out_shape=jax.ShapeDtypeStruct((NP, D), jnp.float32),
)


def _tc_final_body(s_ref, hn_ref, cnt_ref, bat_ref, b2_ref, fcw_ref, fcb_ref,
                   d1w_ref, d1b_ref, d2w_ref, d2b_ref, d3w_ref, d3b_ref,
                   out_ref, pool, cntg, g1s):
    g = pl.program_id(0)
    i = pl.program_id(1)

    @pl.when(i == 0)
    def _():
        pool[...] = jnp.zeros_like(pool)
        cntg[...] = jnp.zeros_like(cntg)

    deg = jnp.sum(cnt_ref[...], axis=1, keepdims=True)
    dinv = lax.rsqrt(deg + 1.0)
    o2 = jnp.maximum((s_ref[...] + hn_ref[...]) * dinv + b2_ref[...], 0.0)
    oh = (bat_ref[...] == lax.broadcasted_iota(jnp.int32, (RB, G), 1)
          ).astype(jnp.float32)
    pool[...] += lax.dot_general(oh, o2, (((0,), (0,)), ((), ())),
                                 preferred_element_type=jnp.float32)
    cntg[...] += jnp.broadcast_to(jnp.sum(oh, axis=0)[:, None], (G, D))

    @pl.when(i == NPB - 1)
    def _():
        gv = pool[...] / jnp.maximum(cntg[...], 1.0)
        fco = jnp.dot(gv, fcw_ref[...],
                      preferred_element_type=jnp.float32) + fcb_ref[...]

        @pl.when(g == 0)
        def _():
            g1s[...] = fco

        @pl.when(g == 1)
        def _():
            gs = jnp.maximum(g1s[...] + fco, 0.0)
            h = jnp.maximum(jnp.dot(gs, d1w_ref[...],
                                    preferred_element_type=jnp.float32)
                            + d1b_ref[...], 0.0)
            h = jnp.maximum(jnp.dot(h, d2w_ref[...],
                                    preferred_element_type=jnp.float32)
                            + d2b_ref[...], 0.0)
            out_ref[...] = jnp.dot(h, d3w_ref[...],
                                   preferred_element_type=jnp.float32) \
                + d3b_ref[...]


_tc_final = pl.pallas_call(
    _tc_final_body,
    grid=(NC, NPB),
    in_specs=[
        pl.BlockSpec((RB, D), lambda g, i: (g * NPB + i, 0)),
        pl.BlockSpec((RB, D), lambda g, i: (g * NPB + i, 0)),
        pl.BlockSpec((RB, NS), lambda g, i: (g * NPB + i, 0)),
        pl.BlockSpec((RB, 1), lambda g, i: (g * NPB + i, 0)),
        pl.BlockSpec((1, D), lambda g, i: (0, 0)),
        pl.BlockSpec((D, D), lambda g, i: (0, 0)),
        pl.BlockSpec((1, D), lambda g, i: (0, 0)),
        pl.BlockSpec((D, D), lambda g, i: (0, 0)),
        pl.BlockSpec((1, D), lambda g, i: (0, 0)),
        pl.BlockSpec((D, D), lambda g, i: (0, 0)),
        pl.BlockSpec((1, D), lambda g, i: (0, 0)),
        pl.BlockSpec((D, D), lambda g, i: (0, 0)),
        pl.BlockSpec((1, D), lambda g, i: (0, 0)),
    ],
    out_specs=pl.BlockSpec((G, D), lambda g, i: (0, 0)),
    out_shape=jax.ShapeDtypeStruct((G, D), jnp.float32),
    scratch_shapes=[
        pltpu.VMEM((G, D), jnp.float32),
        pltpu.VMEM((G, D), jnp.float32),
        pltpu.VMEM((G, D), jnp.float32),
    ],
)


def _prep_edges(ei):
    src = ei[0].astype(jnp.int32)
    dst = ei[1].astype(jnp.int32)
    src = jnp.concatenate([src, jnp.zeros((EPAD - E,), jnp.int32)])
    dst = jnp.concatenate([dst, jnp.full((EPAD - E,), NP - 1, jnp.int32)])
    return src.reshape(NS * NCH, CH), dst.reshape(NS * NCH, CH)


def _pad_nodes(a, pad_val):
    return jnp.concatenate(
        [a.astype(jnp.int32), jnp.full((NP - N,), pad_val, jnp.int32)])


def kernel(x1, edge_index1, batch1, x2, edge_index2, batch2,
           emb, W1, b1, W2, b2, fcW, fcb, d1W, d1b, d2W, d2b, d3W, d3b):
    f32 = jnp.float32
    s1, d1 = _prep_edges(edge_index1)
    s2, d2 = _prep_edges(edge_index2)
    dst_all = jnp.concatenate([d1, d2], 0)

    zeros_rows = jnp.zeros((RPS, D), f32)

    xi1 = _pad_nodes(x1, 0).reshape(NP, 1)
    xi2 = _pad_nodes(x2, 0).reshape(NP, 1)
    bat = jnp.concatenate([_pad_nodes(batch1, G), _pad_nodes(batch2, G)]
                          ).reshape(NC * NP, 1)
    emb_p = jnp.zeros((16, D), f32).at[:11].set(emb)

    sc_degree, sc_conv1, sc_conv2 = _sc_kernels()
    cnt = sc_degree(dst_all, jnp.zeros((NP,), f32))
    cnt3 = cnt.reshape(NC, NS, NP).transpose(0, 2, 1).reshape(NC * NP, NS)
    cnt_g1, cnt_g2 = cnt3[:NP], cnt3[NP:]

    b1r = b1.reshape(1, D)
    hn1_1 = _tc_hn1(cnt_g1, xi1, emb_p, W1)
    hn1_2 = _tc_hn1(cnt_g2, xi2, emb_p, W1)
    sum1_1 = sc_conv1(hn1_1, s1, d1, zeros_rows)
    sum1_2 = sc_conv2(hn1_2, s2, d2, zeros_rows)
    hn2_1 = _tc_hn2(sum1_1, hn1_1, cnt_g1, b1r, W2)
    hn2_2 = _tc_hn2(sum1_2, hn1_2, cnt_g2, b1r, W2)
    sum2_1 = sc_conv1(hn2_1, s1, d1, zeros_rows)
    sum2_2 = sc_conv2(hn2_2, s2, d2, zeros_rows)

    sum2 = jnp.concatenate([sum2_1, sum2_2], 0)
    hn2 = jnp.concatenate([hn2_1, hn2_2], 0)
    return _tc_final(sum2, hn2, cnt3, bat, b2.reshape(1, D), fcW,
                     fcb.reshape(1, D), d1W, d1b.reshape(1, D),
                     d2W, d2b.reshape(1, D), d3W, d3b.reshape(1, D))


# speedup vs baseline: 1.8126x; 1.8126x over previous
import functools

import jax
import jax.numpy as jnp
from jax import lax
from jax.experimental import pallas as pl
from jax.experimental.pallas import tpu as pltpu
from jax.experimental.pallas import tpu_sc as plsc

N = 10000
E = 320000
G = 64
D = 128
NC = 2
NS = 16
CH = 128
NP = 10240
RPS = NP // NS
NCH = 160
IB = 32
ESUB = NCH * CH
EPAD = ESUB * NS
RB = 512
NBLK = NC * NP // RB
NPB = NP // RB
CNB = RB // D


@functools.cache
def _sc_kernels():
    mesh = plsc.VectorSubcoreMesh(core_axis_name="c", subcore_axis_name="s",
                                  num_cores=NC, num_subcores=NS)

    CNR = NP // D

    @functools.partial(
        pl.kernel,
        out_type=jax.ShapeDtypeStruct((NC * NS * NP,), jnp.float32),
        mesh=mesh,
        scratch_types=[
            pltpu.VMEM((NCH, CH), jnp.int32),
            pltpu.VMEM((NP,), jnp.float32),
        ],
        compiler_params=pltpu.CompilerParams(needs_layout_passes=False),
    )
    def _sc_degree(dst_hbm, zeros1d_hbm, cnt_hbm, dst_v, cnt_v):
        cid = lax.axis_index("c")
        sid = lax.axis_index("s")
        wid = cid * NS + sid
        pltpu.sync_copy(zeros1d_hbm, cnt_v)
        pltpu.sync_copy(dst_hbm.at[pl.ds(wid * NCH, NCH)], dst_v)
        ones16 = jnp.ones((16,), jnp.float32)

        def step(e, carry):
            idx = dst_v[e >> 3, pl.ds((e & 7) * 16, 16)]
            plsc.addupdate_scatter(cnt_v, [idx], ones16)
            return carry

        lax.fori_loop(0, NCH * (CH // 16), step, 0)
        pltpu.sync_copy(cnt_v, cnt_hbm.at[pl.ds(wid * NP, NP)])

    @functools.partial(
        pl.kernel,
        out_type=jax.ShapeDtypeStruct((NC * NP, D), jnp.float32),
        mesh=mesh,
        scratch_types=[
            pltpu.VMEM((IB, CH), jnp.int32),
            pltpu.VMEM((IB, CH), jnp.int32),
            pltpu.VMEM((CH, D), jnp.float32),
            pltpu.VMEM((CH, D), jnp.float32),
            pltpu.SemaphoreType.DMA,
            pltpu.SemaphoreType.DMA,
            pltpu.SemaphoreType.DMA,
            pltpu.SemaphoreType.DMA,
            pltpu.SemaphoreType.DMA,
            pltpu.SemaphoreType.DMA,
            pltpu.VMEM_SHARED((NP, D), jnp.float32),
        ],
    )
    def _sc_conv(hn_hbm, src_hbm, dst_hbm, zeros_hbm, out_hbm,
                 src_v, dst_v, rows0, rows1, gs0a, gs0b, gs1a, gs1b,
                 ss0, ss1, acc):
        cid = lax.axis_index("c")
        sid = lax.axis_index("s")
        wid = cid * NS + sid
        pltpu.sync_copy(zeros_hbm, acc.at[pl.ds(sid * RPS, RPS)])
        plsc.subcore_barrier()

        H = CH // 2

        def gather(j, buf, sa, sb):
            pltpu.async_copy(hn_hbm.at[src_v.at[j, pl.ds(0, H)]],
                             buf.at[pl.ds(0, H)], sa)
            pltpu.async_copy(hn_hbm.at[src_v.at[j, pl.ds(H, H)]],
                             buf.at[pl.ds(H, H)], sb)

        def gwait(buf, sa, sb):
            pltpu.make_async_copy(hn_hbm.at[src_v.at[0, pl.ds(0, H)]],
                                  buf.at[pl.ds(0, H)], sa).wait()
            pltpu.make_async_copy(hn_hbm.at[src_v.at[0, pl.ds(H, H)]],
                                  buf.at[pl.ds(H, H)], sb).wait()

        def block(b, carry):
            base = wid * NCH + b * IB
            pltpu.sync_copy(src_hbm.at[pl.ds(base, IB)], src_v)
            pltpu.sync_copy(dst_hbm.at[pl.ds(base, IB)], dst_v)
            gather(0, rows0, gs0a, gs0b)
            gather(1, rows1, gs1a, gs1b)

            def pair(i, c2):
                j0 = 2 * i
                j1 = j0 + 1
                gwait(rows0, gs0a, gs0b)
                s0 = pltpu.async_copy(rows0, acc.at[dst_v.at[j0]], ss0,
                                      add=True)
                gwait(rows1, gs1a, gs1b)
                s1 = pltpu.async_copy(rows1, acc.at[dst_v.at[j1]], ss1,
                                      add=True)

                @pl.when(i + 1 < IB // 2)
                def _():
                    s0.wait()
                    gather(j0 + 2, rows0, gs0a, gs0b)
                    s1.wait()
                    gather(j1 + 2, rows1, gs1a, gs1b)

                @pl.when(i + 1 >= IB // 2)
                def _():
                    s0.wait()
                    s1.wait()

                return c2

            lax.fori_loop(0, IB // 2, pair, 0)
            return carry

        lax.fori_loop(0, NCH // IB, block, 0)
        plsc.subcore_barrier()
        pltpu.sync_copy(acc.at[pl.ds(sid * RPS, RPS)],
                        out_hbm.at[pl.ds(cid * NP + sid * RPS, RPS)])

    return _sc_degree, _sc_conv


def _tc_hn1_body(cnt_ref, xi_ref, emb_ref, w1_ref, out_ref):
    deg = jnp.sum(cnt_ref[...], axis=1, keepdims=True)
    dinv = lax.rsqrt(deg + 1.0)
    oh = (xi_ref[...] == lax.broadcasted_iota(jnp.int32, (RB, 16), 1)
          ).astype(jnp.float32)
    ew = jnp.dot(emb_ref[...], w1_ref[...], preferred_element_type=jnp.float32)
    h = jnp.dot(oh, ew, preferred_element_type=jnp.float32)
    out_ref[...] = h * dinv


_tc_hn1 = pl.pallas_call(
    _tc_hn1_body,
    grid=(NBLK,),
    in_specs=[
        pl.BlockSpec((RB, NS), lambda i: (i, 0)),
        pl.BlockSpec((RB, 1), lambda i: (i, 0)),
        pl.BlockSpec((16, D), lambda i: (0, 0)),
        pl.BlockSpec((D, D), lambda i: (0, 0)),
    ],
    out_specs=pl.BlockSpec((RB, D), lambda i: (i, 0)),
    out_shape=jax.ShapeDtypeStruct((NC * NP, D), jnp.float32),
)


def _tc_hn2_body(s_ref, hn_ref, cnt_ref, b_ref, w_ref, out_ref):
    deg = jnp.sum(cnt_ref[...], axis=1, keepdims=True)
    dinv = lax.rsqrt(deg + 1.0)
    o = jnp.maximum((s_ref[...] + hn_ref[...]) * dinv + b_ref[...], 0.0)
    out_ref[...] = jnp.dot(o, w_ref[...],
                           preferred_element_type=jnp.float32) * dinv


_tc_hn2 = pl.pallas_call(
    _tc_hn2_body,
    grid=(NBLK,),
    in_specs=[
        pl.BlockSpec((RB, D), lambda i: (i, 0)),
        pl.BlockSpec((RB, D), lambda i: (i, 0)),
        pl.BlockSpec((RB, NS), lambda i: (i, 0)),
        pl.BlockSpec((1, D), lambda i: (0, 0)),
        pl.BlockSpec((D, D), lambda i: (0, 0)),
    ],
    out_specs=pl.BlockSpec((RB, D), lambda i: (i, 0)),
    out_shape=jax.ShapeDtypeStruct((NC * NP, D), jnp.float32),
)


def _tc_final_body(s_ref, hn_ref, cnt_ref, bat_ref, b2_ref, fcw_ref, fcb_ref,
                   d1w_ref, d1b_ref, d2w_ref, d2b_ref, d3w_ref, d3b_ref,
                   out_ref, pool, cntg, g1s):
    g = pl.program_id(0)
    i = pl.program_id(1)

    @pl.when(i == 0)
    def _():
        pool[...] = jnp.zeros_like(pool)
        cntg[...] = jnp.zeros_like(cntg)

    deg = jnp.sum(cnt_ref[...], axis=1, keepdims=True)
    dinv = lax.rsqrt(deg + 1.0)
    o2 = jnp.maximum((s_ref[...] + hn_ref[...]) * dinv + b2_ref[...], 0.0)
    oh = (bat_ref[...] == lax.broadcasted_iota(jnp.int32, (RB, G), 1)
          ).astype(jnp.float32)
    pool[...] += lax.dot_general(oh, o2, (((0,), (0,)), ((), ())),
                                 preferred_element_type=jnp.float32)
    cntg[...] += jnp.broadcast_to(jnp.sum(oh, axis=0)[:, None], (G, D))

    @pl.when(i == NPB - 1)
    def _():
        gv = pool[...] / jnp.maximum(cntg[...], 1.0)
        fco = jnp.dot(gv, fcw_ref[...],
                      preferred_element_type=jnp.float32) + fcb_ref[...]

        @pl.when(g == 0)
        def _():
            g1s[...] = fco

        @pl.when(g == 1)
        def _():
            gs = jnp.maximum(g1s[...] + fco, 0.0)
            h = jnp.maximum(jnp.dot(gs, d1w_ref[...],
                                    preferred_element_type=jnp.float32)
                            + d1b_ref[...], 0.0)
            h = jnp.maximum(jnp.dot(h, d2w_ref[...],
                                    preferred_element_type=jnp.float32)
                            + d2b_ref[...], 0.0)
            out_ref[...] = jnp.dot(h, d3w_ref[...],
                                   preferred_element_type=jnp.float32) \
                + d3b_ref[...]


_tc_final = pl.pallas_call(
    _tc_final_body,
    grid=(NC, NPB),
    in_specs=[
        pl.BlockSpec((RB, D), lambda g, i: (g * NPB + i, 0)),
        pl.BlockSpec((RB, D), lambda g, i: (g * NPB + i, 0)),
        pl.BlockSpec((RB, NS), lambda g, i: (g * NPB + i, 0)),
        pl.BlockSpec((RB, 1), lambda g, i: (g * NPB + i, 0)),
        pl.BlockSpec((1, D), lambda g, i: (0, 0)),
        pl.BlockSpec((D, D), lambda g, i: (0, 0)),
        pl.BlockSpec((1, D), lambda g, i: (0, 0)),
        pl.BlockSpec((D, D), lambda g, i: (0, 0)),
        pl.BlockSpec((1, D), lambda g, i: (0, 0)),
        pl.BlockSpec((D, D), lambda g, i: (0, 0)),
        pl.BlockSpec((1, D), lambda g, i: (0, 0)),
        pl.BlockSpec((D, D), lambda g, i: (0, 0)),
        pl.BlockSpec((1, D), lambda g, i: (0, 0)),
    ],
    out_specs=pl.BlockSpec((G, D), lambda g, i: (0, 0)),
    out_shape=jax.ShapeDtypeStruct((G, D), jnp.float32),
    scratch_shapes=[
        pltpu.VMEM((G, D), jnp.float32),
        pltpu.VMEM((G, D), jnp.float32),
        pltpu.VMEM((G, D), jnp.float32),
    ],
)


def _prep_edges(ei, goff):
    src = ei[0].astype(jnp.int32)
    dst = ei[1].astype(jnp.int32)
    src = jnp.concatenate([src, jnp.zeros((EPAD - E,), jnp.int32)]) + goff
    dst = jnp.concatenate([dst, jnp.full((EPAD - E,), NP - 1, jnp.int32)])
    return src.reshape(NS * NCH, CH), dst.reshape(NS * NCH, CH)


def _pad_nodes(a, pad_val):
    return jnp.concatenate(
        [a.astype(jnp.int32), jnp.full((NP - N,), pad_val, jnp.int32)])


def kernel(x1, edge_index1, batch1, x2, edge_index2, batch2,
           emb, W1, b1, W2, b2, fcW, fcb, d1W, d1b, d2W, d2b, d3W, d3b):
    f32 = jnp.float32
    s1, d1 = _prep_edges(edge_index1, 0)
    s2, d2 = _prep_edges(edge_index2, NP)
    src_all = jnp.concatenate([s1, s2], 0)
    dst_all = jnp.concatenate([d1, d2], 0)

    zeros_rows = jnp.zeros((RPS, D), f32)

    xi = jnp.concatenate([_pad_nodes(x1, 0), _pad_nodes(x2, 0)]
                         ).reshape(NC * NP, 1)
    bat = jnp.concatenate([_pad_nodes(batch1, G), _pad_nodes(batch2, G)]
                          ).reshape(NC * NP, 1)
    emb_p = jnp.zeros((16, D), f32).at[:11].set(emb)

    sc_degree, sc_conv = _sc_kernels()
    cnt = sc_degree(dst_all, jnp.zeros((NP,), f32))
    cnt3 = cnt.reshape(NC, NS, NP).transpose(0, 2, 1).reshape(NC * NP, NS)
    hn1 = _tc_hn1(cnt3, xi, emb_p, W1)
    sum1 = sc_conv(hn1, src_all, dst_all, zeros_rows)
    hn2 = _tc_hn2(sum1, hn1, cnt3, b1.reshape(1, D), W2)
    sum2 = sc_conv(hn2, src_all, dst_all, zeros_rows)
    return _tc_final(sum2, hn2, cnt3, bat, b2.reshape(1, D), fcW,
                     fcb.reshape(1, D), d1W, d1b.reshape(1, D),
                     d2W, d2b.reshape(1, D), d3W, d3b.reshape(1, D))
